# baseline (speedup 1.0000x reference)
import functools

import jax
import jax.numpy as jnp
from jax import lax
from jax.experimental import pallas as pl
from jax.experimental.pallas import tpu as pltpu
from jax.experimental.pallas import tpu_sc as plsc

_NC = 2
_NS = 16
_LN_EPS = 1e-5


def _mlp_body(packed, ln, has_resid, *refs):
    nparts = len(packed)
    nw = nparts + sum(packed)
    parts = refs[:nparts]
    wrefs = iter(refs[nparts:nparts + nw])
    w0s = [(next(wrefs), next(wrefs)) if f else next(wrefs) for f in packed]
    b0, w1, b1, w2, b2 = refs[nparts + nw:nparts + nw + 5]
    resid_ref = refs[nparts + nw + 5] if has_resid else None
    out_ref = refs[-1]

    acc = None
    for p, w in zip(parts, w0s):
        xv = p[...]
        if xv.ndim == 3:
            xv = xv[0]
        if xv.dtype != jnp.float32:
            xv = xv.astype(jnp.float32)
        if isinstance(w, tuple):
            w_ev, w_od = w
            u = lax.bitcast_convert_type(xv, jnp.uint32)
            v_ev = lax.bitcast_convert_type(u << 16, jnp.float32)
            v_od = lax.bitcast_convert_type(u & jnp.uint32(0xFFFF0000),
                                            jnp.float32)
            t = (jnp.dot(v_ev, w_ev[...], preferred_element_type=jnp.float32)
                 + jnp.dot(v_od, w_od[...],
                           preferred_element_type=jnp.float32))
        else:
            t = jnp.dot(xv, w[...], preferred_element_type=jnp.float32)
        acc = t if acc is None else acc + t
    h = jnp.maximum(acc + b0[...], 0.0)
    h = jnp.maximum(jnp.dot(h, w1[...], preferred_element_type=jnp.float32)
                    + b1[...], 0.0)
    y = jnp.dot(h, w2[...], preferred_element_type=jnp.float32) + b2[...]
    if ln:
        m = jnp.mean(y, axis=-1, keepdims=True)
        yc = y - m
        v = jnp.mean(yc * yc, axis=-1, keepdims=True)
        y = yc * lax.rsqrt(v + _LN_EPS)
    if has_resid:
        r = resid_ref[...]
        if r.ndim == 3:
            r = r[0]
        y = y + r
    out_ref[...] = y


def _full_spec(arr):
    nd = arr.ndim
    return pl.BlockSpec(arr.shape, lambda i, _n=nd: (0,) * _n)


def _mlp(parts, layers, *, ln, resid=None, block, out_rows=None):
    arrays = [a for a, _ in parts]
    specs = [s(block) for _, s in parts]
    (w0_list, b0), (w1, b1), (w2, b2) = layers
    n_rows = parts[0][1].n_rows
    out_dim = w2.shape[1]
    b0 = b0.reshape(1, -1)
    b1 = b1.reshape(1, -1)
    b2 = b2.reshape(1, -1)
    packed = [isinstance(w, tuple) for w in w0_list]
    flat_w0 = []
    for w in w0_list:
        flat_w0.extend(w if isinstance(w, tuple) else (w,))
    inputs = list(arrays) + flat_w0 + [b0, w1, b1, w2, b2]
    in_specs = list(specs) + [_full_spec(w) for w in flat_w0] + [
        _full_spec(b0), _full_spec(w1), _full_spec(b1),
        _full_spec(w2), _full_spec(b2)]
    if resid is not None:
        arr, sfn = resid
        inputs.append(arr)
        in_specs.append(sfn(block))
    return pl.pallas_call(
        functools.partial(_mlp_body, packed, ln, resid is not None),
        grid=(n_rows // block,),
        in_specs=in_specs,
        out_specs=pl.BlockSpec((block, out_dim), lambda i: (i, 0)),
        out_shape=jax.ShapeDtypeStruct((out_rows or n_rows, out_dim),
                                       jnp.float32),
    )(*inputs)


def _rows2d(arr):
    d = arr.shape[1]

    def f(block):
        return pl.BlockSpec((block, d), lambda i: (i, 0))
    f.n_rows = arr.shape[0]
    return arr, f


def _rows2d_view(arr, row_offset, n_rows):
    d = arr.shape[1]

    def f(block):
        o = row_offset // block
        return pl.BlockSpec((block, d), lambda i, o=o: (o + i, 0))
    f.n_rows = n_rows
    return arr, f


def _rows3d(arr, major_idx):
    d = arr.shape[2]

    def f(block):
        return pl.BlockSpec((1, block, d), lambda i, m=major_idx: (m, i, 0))
    f.n_rows = arr.shape[1]
    return arr, f


_GW = 256


def _sc_gather(table, idx2d):
    nworkers, per_w = idx2d.shape
    d = table.shape[1]
    kw = per_w // _GW
    mesh = plsc.VectorSubcoreMesh(core_axis_name="core",
                                  subcore_axis_name="subcore")

    @functools.partial(
        pl.kernel,
        out_type=jax.ShapeDtypeStruct((nworkers * per_w, d), table.dtype),
        mesh=mesh,
        scratch_types=[
            pltpu.VMEM((per_w,), jnp.int32),
            pltpu.VMEM((_GW, d), jnp.float32),
            pltpu.VMEM((_GW, d), jnp.float32),
            pltpu.SemaphoreType.DMA,
            pltpu.SemaphoreType.DMA,
            pltpu.SemaphoreType.DMA,
            pltpu.SemaphoreType.DMA,
        ])
    def k(x_hbm, i_hbm, o_hbm, idx_v, buf0, buf1, gs0, gs1, ws0, ws1):
        cid = lax.axis_index("core")
        sid = lax.axis_index("subcore")
        wid = sid * _NC + cid
        base = wid * per_w

        pltpu.sync_copy(i_hbm.at[wid], idx_v)

        @pl.loop(0, kw, step=2)
        def _win(j):
            @pl.when(j > 0)
            def _():
                pltpu.make_async_copy(
                    buf0, o_hbm.at[pl.ds(base, _GW)], ws0).wait()
                pltpu.make_async_copy(
                    buf1, o_hbm.at[pl.ds(base, _GW)], ws1).wait()
            c0 = pltpu.make_async_copy(
                x_hbm.at[idx_v.at[pl.ds(j * _GW, _GW)]], buf0, gs0)
            c0.start()
            c1 = pltpu.make_async_copy(
                x_hbm.at[idx_v.at[pl.ds((j + 1) * _GW, _GW)]], buf1, gs1)
            c1.start()
            c0.wait()
            pltpu.make_async_copy(
                buf0, o_hbm.at[pl.ds(base + j * _GW, _GW)], ws0).start()
            c1.wait()
            pltpu.make_async_copy(
                buf1, o_hbm.at[pl.ds(base + (j + 1) * _GW, _GW)], ws1).start()

        pltpu.make_async_copy(buf0, o_hbm.at[pl.ds(base, _GW)], ws0).wait()
        pltpu.make_async_copy(buf1, o_hbm.at[pl.ds(base, _GW)], ws1).wait()

    return k(table, idx2d)


_SW = 128


def _sc_segment_sum(vals, idx3d, acc_rows):
    d = vals.shape[1]
    k_chunks = idx3d.shape[1]
    per_w = k_chunks * _SW
    rows_per_sub = acc_rows // _NS
    zr = 128
    nz = rows_per_sub // zr
    mesh = plsc.VectorSubcoreMesh(core_axis_name="core",
                                  subcore_axis_name="subcore")

    @functools.partial(
        pl.kernel,
        out_type=jax.ShapeDtypeStruct((_NC, acc_rows, d), jnp.float32),
        mesh=mesh,
        scratch_types=[
            pltpu.VMEM((k_chunks, _SW), jnp.int32),
            pltpu.VMEM((_SW, d), jnp.float32),
            pltpu.VMEM((zr, d), jnp.float32),
            pltpu.VMEM_SHARED((acc_rows, d), jnp.float32),
        ])
    def k(vals_hbm, idx_hbm, out_hbm, idx_v, rows_v, zbuf, acc):
        cid = lax.axis_index("core")
        sid = lax.axis_index("subcore")
        wid = sid * _NC + cid

        zbuf[...] = jnp.zeros_like(zbuf)

        @pl.loop(0, nz)
        def _zero(z):
            pltpu.sync_copy(zbuf,
                            acc.at[pl.ds(sid * rows_per_sub + z * zr, zr)])

        plsc.subcore_barrier()

        pltpu.sync_copy(idx_hbm.at[wid], idx_v)

        @pl.loop(0, k_chunks)
        def _scat(j):
            base = wid * per_w + j * _SW
            pltpu.sync_copy(vals_hbm.at[pl.ds(base, _SW)], rows_v)
            pltpu.sync_copy(rows_v, acc.at[idx_v.at[j]], add=True)

        plsc.subcore_barrier()

        @pl.loop(0, nz)
        def _out(z):
            r = sid * rows_per_sub + z * zr
            pltpu.sync_copy(acc.at[pl.ds(r, zr)],
                            out_hbm.at[cid].at[pl.ds(r, zr)])

    return k(vals, idx3d)


_EDGE_BLOCK = 3200
_NODE_BLOCK = 2000


def _split_first(layers, widths):
    (w0, b0), l1, l2 = layers
    parts = []
    o = 0
    for w in widths:
        parts.append(w0[o:o + w])
        o += w
    return [(parts, b0), l1, l2]


def _pack_bf16(a):
    n, d2 = a.shape
    b = a.astype(jnp.bfloat16).reshape(n, d2 // 2, 2)
    return lax.bitcast_convert_type(b, jnp.float32)


def kernel(x, edge_attr, receivers, senders, params):
    n_nodes = x.shape[0]
    n_edges = receivers.shape[0]
    lat = params["enc_node"][-1][0].shape[1]

    nw = _NC * _NS
    g_pad = -(-2 * n_edges // (nw * _GW * 2)) * (nw * _GW * 2)
    gather_idx = jnp.concatenate(
        [receivers, senders,
         jnp.zeros((g_pad - 2 * n_edges,), jnp.int32)]).reshape(
             nw, g_pad // nw)
    e_pad = -(-n_edges // (nw * _SW)) * (nw * _SW)
    acc_rows = -(-(n_nodes + 1) // (_NS * 128)) * (_NS * 128)
    scat_idx = jnp.concatenate(
        [receivers,
         jnp.full((e_pad - n_edges,), n_nodes, jnp.int32)]).reshape(
             nw, e_pad // (nw * _SW), _SW)

    enc_node = _split_first(params["enc_node"], [x.shape[1]])
    enc_edge = _split_first(params["enc_edge"], [edge_attr.shape[1]])
    dec = _split_first(params["dec"], [lat])

    x_lat = _mlp([_rows2d(x)], enc_node, ln=True, block=_NODE_BLOCK)
    e_lat = _mlp([_rows2d(edge_attr)], enc_edge, ln=True, block=_EDGE_BLOCK)

    pre_x = x_lat
    upd_e_prev = None

    for p in params["proc"]:
        (w0e, b0e), l1e, l2e = _split_first(p["edge"], [lat, lat, lat])
        (w0n, b0n), l1n, l2n = _split_first(p["node"], [lat, lat])

        g = _sc_gather(pre_x, gather_idx)

        eparts = [_rows2d_view(e_lat, 0, n_edges)]
        ew0 = [w0e[0]]
        if upd_e_prev is not None:
            eparts.append(_rows2d_view(upd_e_prev, 0, n_edges))
            ew0.append(w0e[0])
        eparts.append(_rows2d_view(g, 0, n_edges))
        ew0.append(w0e[1])
        eparts.append(_rows2d_view(g, n_edges, n_edges))
        ew0.append(w0e[2])
        upd_e = _mlp(eparts, [(ew0, b0e), l1e, l2e], ln=True,
                     block=_EDGE_BLOCK, out_rows=e_pad)

        partials = _sc_segment_sum(upd_e, scat_idx, acc_rows)

        nparts = [_rows2d(pre_x), _rows3d(partials, 0), _rows3d(partials, 1)]
        nw0 = [w0n[0], w0n[1], w0n[1]]
        pre_x = _mlp(nparts, [(nw0, b0n), l1n, l2n], ln=True,
                     resid=_rows2d(x_lat), block=_NODE_BLOCK)
        upd_e_prev = upd_e

    return _mlp([_rows2d(pre_x)], dec, ln=False, block=_NODE_BLOCK)


# speedup vs baseline: 1.7257x; 1.7257x over previous
import functools

import jax
import jax.numpy as jnp
from jax import lax
from jax.experimental import pallas as pl
from jax.experimental.pallas import tpu as pltpu
from jax.experimental.pallas import tpu_sc as plsc

_NC = 2
_NS = 16
_LN_EPS = 1e-5


def _mlp_body(kinds, ln, has_resid, first_only, *refs):
    nparts = len(kinds)
    nw = sum(1 for f in kinds if f == 0)
    parts = refs[:nparts]
    wrefs = iter(refs[nparts:nparts + nw])
    w0s = [None if f == 2 else next(wrefs) for f in kinds]
    b0, w1, b1, w2, b2 = refs[nparts + nw:nparts + nw + 5]
    resid_ref = refs[nparts + nw + 5] if has_resid else None
    out_ref = refs[-1]

    acc = None
    for p, w in zip(parts, w0s):
        xv = p[...]
        if xv.ndim == 3:
            xv = xv[0]
        if xv.dtype != jnp.float32:
            xv = xv.astype(jnp.float32)
        if w is None:
            t = xv
        else:
            t = jnp.dot(xv, w[...], preferred_element_type=jnp.float32)
        acc = t if acc is None else acc + t
    y = acc + b0[...]
    if not first_only:
        h = jnp.maximum(y, 0.0)
        h = jnp.maximum(
            jnp.dot(h, w1[...], preferred_element_type=jnp.float32)
            + b1[...], 0.0)
        y = jnp.dot(h, w2[...], preferred_element_type=jnp.float32) + b2[...]
    if ln:
        m = jnp.mean(y, axis=-1, keepdims=True)
        yc = y - m
        v = jnp.mean(yc * yc, axis=-1, keepdims=True)
        y = yc * lax.rsqrt(v + _LN_EPS)
    if has_resid:
        r = resid_ref[...]
        if r.ndim == 3:
            r = r[0]
        y = y + r
    out_ref[...] = y


def _full_spec(arr):
    nd = arr.ndim
    return pl.BlockSpec(arr.shape, lambda i, _n=nd: (0,) * _n)


def _mlp(parts, layers, *, ln, resid=None, block, out_rows=None,
         first_only=False):
    arrays = [a for a, _ in parts]
    specs = [s(block) for _, s in parts]
    (w0_list, b0), (w1, b1), (w2, b2) = layers
    n_rows = parts[0][1].n_rows
    out_dim = b0.size if first_only else w2.shape[1]
    b0 = b0.reshape(1, -1)
    b1 = b1.reshape(1, -1)
    b2 = b2.reshape(1, -1)
    kinds = [2 if w is None else 0 for w in w0_list]
    flat_w0 = [w for w in w0_list if w is not None]
    inputs = list(arrays) + flat_w0 + [b0, w1, b1, w2, b2]
    in_specs = list(specs) + [_full_spec(w) for w in flat_w0] + [
        _full_spec(b0), _full_spec(w1), _full_spec(b1),
        _full_spec(w2), _full_spec(b2)]
    if resid is not None:
        arr, sfn = resid
        inputs.append(arr)
        in_specs.append(sfn(block))
    return pl.pallas_call(
        functools.partial(_mlp_body, kinds, ln, resid is not None,
                          first_only),
        grid=(n_rows // block,),
        in_specs=in_specs,
        out_specs=pl.BlockSpec((block, out_dim), lambda i: (i, 0)),
        out_shape=jax.ShapeDtypeStruct((out_rows or n_rows, out_dim),
                                       jnp.float32),
    )(*inputs)


def _rows2d(arr):
    d = arr.shape[1]

    def f(block):
        return pl.BlockSpec((block, d), lambda i: (i, 0))
    f.n_rows = arr.shape[0]
    return arr, f


def _rows2d_view(arr, row_offset, n_rows):
    d = arr.shape[1]

    def f(block):
        o = row_offset // block
        return pl.BlockSpec((block, d), lambda i, o=o: (o + i, 0))
    f.n_rows = n_rows
    return arr, f


def _rows3d(arr, major_idx):
    d = arr.shape[2]

    def f(block):
        return pl.BlockSpec((1, block, d), lambda i, m=major_idx: (m, i, 0))
    f.n_rows = arr.shape[1]
    return arr, f


_GW = 128


def _sc_gather(table, idx2d):
    n = idx2d.shape[1]
    d = table.shape[1]
    mesh = plsc.VectorSubcoreMesh(core_axis_name="core",
                                  subcore_axis_name="subcore")

    @functools.partial(
        pl.kernel,
        out_type=jax.ShapeDtypeStruct((n, d), table.dtype),
        mesh=mesh)
    def k(x_hbm, i_hbm, o_hbm):
        def body(i_vmem, o_vmem):
            pltpu.sync_copy(x_hbm.at[i_vmem.at[0]], o_vmem)

        pltpu.emit_pipeline(
            body,
            grid=(n // _GW,),
            in_specs=[pl.BlockSpec((1, _GW), lambda i: (0, i))],
            out_specs=[pl.BlockSpec((_GW, d), lambda i: (i, 0))],
            core_axis_name=("core", "subcore"),
            dimension_semantics=(pltpu.PARALLEL,),
        )(i_hbm, o_hbm)

    return k(table, idx2d)


_SW = 128


def _sc_segment_sum(vals, idx3d, acc_rows):
    d = vals.shape[1]
    k_chunks = idx3d.shape[1]
    per_w = k_chunks * _SW
    rows_per_sub = acc_rows // _NS
    zr = 128
    nz = rows_per_sub // zr
    mesh = plsc.VectorSubcoreMesh(core_axis_name="core",
                                  subcore_axis_name="subcore")

    @functools.partial(
        pl.kernel,
        out_type=jax.ShapeDtypeStruct((_NC, acc_rows, d), jnp.float32),
        mesh=mesh,
        scratch_types=[
            pltpu.VMEM((k_chunks, _SW), jnp.int32),
            pltpu.VMEM((_SW, d), jnp.float32),
            pltpu.VMEM((zr, d), jnp.float32),
            pltpu.VMEM_SHARED((acc_rows, d), jnp.float32),
        ])
    def k(vals_hbm, idx_hbm, out_hbm, idx_v, rows_v, zbuf, acc):
        cid = lax.axis_index("core")
        sid = lax.axis_index("subcore")
        wid = sid * _NC + cid

        zbuf[...] = jnp.zeros_like(zbuf)

        @pl.loop(0, nz)
        def _zero(z):
            pltpu.sync_copy(zbuf,
                            acc.at[pl.ds(sid * rows_per_sub + z * zr, zr)])

        plsc.subcore_barrier()

        pltpu.sync_copy(idx_hbm.at[wid], idx_v)

        @pl.loop(0, k_chunks)
        def _scat(j):
            base = wid * per_w + j * _SW
            pltpu.sync_copy(vals_hbm.at[pl.ds(base, _SW)], rows_v)
            pltpu.sync_copy(rows_v, acc.at[idx_v.at[j]], add=True)

        plsc.subcore_barrier()

        @pl.loop(0, nz)
        def _out(z):
            r = sid * rows_per_sub + z * zr
            pltpu.sync_copy(acc.at[pl.ds(r, zr)],
                            out_hbm.at[cid].at[pl.ds(r, zr)])

    return k(vals, idx3d)


_EDGE_BLOCK = 3200
_NODE_BLOCK = 2000


def _split_first(layers, widths):
    (w0, b0), l1, l2 = layers
    parts = []
    o = 0
    for w in widths:
        parts.append(w0[o:o + w])
        o += w
    return [(parts, b0), l1, l2]


def _pack_bf16(a):
    n, d2 = a.shape
    b = a.astype(jnp.bfloat16).reshape(n, d2 // 2, 2)
    return lax.bitcast_convert_type(b, jnp.float32)


def kernel(x, edge_attr, receivers, senders, params):
    n_nodes = x.shape[0]
    n_edges = receivers.shape[0]
    lat = params["enc_node"][-1][0].shape[1]

    nw = _NC * _NS
    g_pad = -(-2 * n_edges // (nw * _GW)) * (nw * _GW)
    gather_idx = jnp.concatenate(
        [receivers, senders,
         jnp.zeros((g_pad - 2 * n_edges,), jnp.int32)]).reshape(1, g_pad)
    e_pad = -(-n_edges // (nw * _SW)) * (nw * _SW)
    acc_rows = -(-(n_nodes + 1) // (_NS * 128)) * (_NS * 128)
    scat_idx = jnp.concatenate(
        [receivers,
         jnp.full((e_pad - n_edges,), n_nodes, jnp.int32)]).reshape(
             nw, e_pad // (nw * _SW), _SW)

    enc_node = _split_first(params["enc_node"], [x.shape[1]])
    enc_edge = _split_first(params["enc_edge"], [edge_attr.shape[1]])
    dec = _split_first(params["dec"], [lat])

    x_lat = _mlp([_rows2d(x)], enc_node, ln=True, block=_NODE_BLOCK)
    e_lat = _mlp([_rows2d(edge_attr)], enc_edge, ln=True, block=_EDGE_BLOCK)

    pre_x = x_lat
    upd_e_prev = None

    for p in params["proc"]:
        (w0e, b0e), l1e, l2e = _split_first(p["edge"], [lat, lat, lat])
        (w0n, b0n), l1n, l2n = _split_first(p["node"], [lat, lat])

        g = _sc_gather(pre_x, gather_idx)

        k1_parts = [_rows2d_view(e_lat, 0, n_edges)]
        k1_w = [w0e[0]]
        if upd_e_prev is not None:
            k1_parts.append(_rows2d_view(upd_e_prev, 0, n_edges))
            k1_w.append(w0e[0])
        h0p = _mlp(k1_parts, [(k1_w, b0e), l1e, l2e], ln=False,
                   block=_EDGE_BLOCK, first_only=True)
        eparts = [_rows2d_view(h0p, 0, n_edges),
                  _rows2d_view(g, 0, n_edges),
                  _rows2d_view(g, n_edges, n_edges)]
        ew0 = [None, w0e[1], w0e[2]]
        upd_e = _mlp(eparts, [(ew0, jnp.zeros_like(b0e)), l1e, l2e],
                     ln=True, block=_EDGE_BLOCK, out_rows=e_pad)

        partials = _sc_segment_sum(upd_e, scat_idx, acc_rows)

        nparts = [_rows2d(pre_x), _rows3d(partials, 0), _rows3d(partials, 1)]
        nw0 = [w0n[0], w0n[1], w0n[1]]
        pre_x = _mlp(nparts, [(nw0, b0n), l1n, l2n], ln=True,
                     resid=_rows2d(x_lat), block=_NODE_BLOCK)
        upd_e_prev = upd_e

    return _mlp([_rows2d(pre_x)], dec, ln=False, block=_NODE_BLOCK)


# speedup vs baseline: 1.8772x; 1.0878x over previous
import functools

import jax
import jax.numpy as jnp
from jax import lax
from jax.experimental import pallas as pl
from jax.experimental.pallas import tpu as pltpu
from jax.experimental.pallas import tpu_sc as plsc

_NC = 2
_NS = 16
_LN_EPS = 1e-5


def _mlp_body(kinds, ln, has_resid, first_only, *refs):
    nparts = len(kinds)
    nw = sum(1 for f in kinds if f == 0)
    parts = refs[:nparts]
    wrefs = iter(refs[nparts:nparts + nw])
    w0s = [None if f == 2 else next(wrefs) for f in kinds]
    b0, w1, b1, w2, b2 = refs[nparts + nw:nparts + nw + 5]
    resid_ref = refs[nparts + nw + 5] if has_resid else None
    out_ref = refs[-1]

    acc = None
    for p, w in zip(parts, w0s):
        xv = p[...]
        if xv.ndim == 3:
            xv = xv[0]
        if xv.dtype != jnp.float32:
            xv = xv.astype(jnp.float32)
        if w is None:
            t = xv
        else:
            t = jnp.dot(xv, w[...], preferred_element_type=jnp.float32)
        acc = t if acc is None else acc + t
    y = acc + b0[...]
    if not first_only:
        h = jnp.maximum(y, 0.0)
        h = jnp.maximum(
            jnp.dot(h, w1[...], preferred_element_type=jnp.float32)
            + b1[...], 0.0)
        y = jnp.dot(h, w2[...], preferred_element_type=jnp.float32) + b2[...]
    if ln:
        m = jnp.mean(y, axis=-1, keepdims=True)
        yc = y - m
        v = jnp.mean(yc * yc, axis=-1, keepdims=True)
        y = yc * lax.rsqrt(v + _LN_EPS)
    if has_resid:
        r = resid_ref[...]
        if r.ndim == 3:
            r = r[0]
        y = y + r
    out_ref[...] = y


def _full_spec(arr):
    nd = arr.ndim
    return pl.BlockSpec(arr.shape, lambda i, _n=nd: (0,) * _n)


def _mlp(parts, layers, *, ln, resid=None, block, out_rows=None,
         first_only=False):
    arrays = [a for a, _ in parts]
    specs = [s(block) for _, s in parts]
    (w0_list, b0), (w1, b1), (w2, b2) = layers
    n_rows = parts[0][1].n_rows
    out_dim = b0.size if first_only else w2.shape[1]
    b0 = b0.reshape(1, -1)
    b1 = b1.reshape(1, -1)
    b2 = b2.reshape(1, -1)
    kinds = [2 if w is None else 0 for w in w0_list]
    flat_w0 = [w for w in w0_list if w is not None]
    inputs = list(arrays) + flat_w0 + [b0, w1, b1, w2, b2]
    in_specs = list(specs) + [_full_spec(w) for w in flat_w0] + [
        _full_spec(b0), _full_spec(w1), _full_spec(b1),
        _full_spec(w2), _full_spec(b2)]
    if resid is not None:
        arr, sfn = resid
        inputs.append(arr)
        in_specs.append(sfn(block))
    return pl.pallas_call(
        functools.partial(_mlp_body, kinds, ln, resid is not None,
                          first_only),
        grid=(n_rows // block,),
        in_specs=in_specs,
        out_specs=pl.BlockSpec((block, out_dim), lambda i: (i, 0)),
        out_shape=jax.ShapeDtypeStruct((out_rows or n_rows, out_dim),
                                       jnp.float32),
    )(*inputs)


def _project_pair(x, wa, wb, block):
    n, d = x.shape
    w = jnp.stack([wa, wb])
    nb = n // block

    def body(x_ref, w_ref, o_ref):
        o_ref[...] = jnp.dot(x_ref[...], w_ref[0],
                             preferred_element_type=jnp.float32)

    return pl.pallas_call(
        body,
        grid=(2 * nb,),
        in_specs=[pl.BlockSpec((block, d), lambda i: (i % nb, 0)),
                  pl.BlockSpec((1,) + wa.shape, lambda i: (i // nb, 0, 0))],
        out_specs=pl.BlockSpec((block, wa.shape[1]), lambda i: (i, 0)),
        out_shape=jax.ShapeDtypeStruct((2 * n, wa.shape[1]), jnp.float32),
    )(x, w)


def _rows2d(arr):
    d = arr.shape[1]

    def f(block):
        return pl.BlockSpec((block, d), lambda i: (i, 0))
    f.n_rows = arr.shape[0]
    return arr, f


def _rows2d_view(arr, row_offset, n_rows):
    d = arr.shape[1]

    def f(block):
        o = row_offset // block
        return pl.BlockSpec((block, d), lambda i, o=o: (o + i, 0))
    f.n_rows = n_rows
    return arr, f


def _rows3d(arr, major_idx):
    d = arr.shape[2]

    def f(block):
        return pl.BlockSpec((1, block, d), lambda i, m=major_idx: (m, i, 0))
    f.n_rows = arr.shape[1]
    return arr, f


_GW = 128


def _sc_gather(table, idx2d):
    n = idx2d.shape[1]
    d = table.shape[1]
    mesh = plsc.VectorSubcoreMesh(core_axis_name="core",
                                  subcore_axis_name="subcore")

    @functools.partial(
        pl.kernel,
        out_type=jax.ShapeDtypeStruct((n, d), table.dtype),
        mesh=mesh)
    def k(x_hbm, i_hbm, o_hbm):
        def body(i_vmem, o_vmem):
            pltpu.sync_copy(x_hbm.at[i_vmem.at[0]], o_vmem)

        pltpu.emit_pipeline(
            body,
            grid=(n // _GW,),
            in_specs=[pl.BlockSpec((1, _GW), lambda i: (0, i))],
            out_specs=[pl.BlockSpec((_GW, d), lambda i: (i, 0))],
            core_axis_name=("core", "subcore"),
            dimension_semantics=(pltpu.PARALLEL,),
        )(i_hbm, o_hbm)

    return k(table, idx2d)


_SW = 128


def _sc_segment_sum(vals, idx3d, acc_rows):
    d = vals.shape[1]
    k_chunks = idx3d.shape[1]
    per_w = k_chunks * _SW
    rows_per_sub = acc_rows // _NS
    zr = 128
    nz = rows_per_sub // zr
    mesh = plsc.VectorSubcoreMesh(core_axis_name="core",
                                  subcore_axis_name="subcore")

    @functools.partial(
        pl.kernel,
        out_type=jax.ShapeDtypeStruct((_NC, acc_rows, d), jnp.float32),
        mesh=mesh,
        scratch_types=[
            pltpu.VMEM((k_chunks, _SW), jnp.int32),
            pltpu.VMEM((_SW, d), jnp.float32),
            pltpu.VMEM((zr, d), jnp.float32),
            pltpu.VMEM_SHARED((acc_rows, d), jnp.float32),
        ])
    def k(vals_hbm, idx_hbm, out_hbm, idx_v, rows_v, zbuf, acc):
        cid = lax.axis_index("core")
        sid = lax.axis_index("subcore")
        wid = sid * _NC + cid

        zbuf[...] = jnp.zeros_like(zbuf)

        @pl.loop(0, nz)
        def _zero(z):
            pltpu.sync_copy(zbuf,
                            acc.at[pl.ds(sid * rows_per_sub + z * zr, zr)])

        plsc.subcore_barrier()

        pltpu.sync_copy(idx_hbm.at[wid], idx_v)

        @pl.loop(0, k_chunks)
        def _scat(j):
            base = wid * per_w + j * _SW
            pltpu.sync_copy(vals_hbm.at[pl.ds(base, _SW)], rows_v)
            pltpu.sync_copy(rows_v, acc.at[idx_v.at[j]], add=True)

        plsc.subcore_barrier()

        @pl.loop(0, nz)
        def _out(z):
            r = sid * rows_per_sub + z * zr
            pltpu.sync_copy(acc.at[pl.ds(r, zr)],
                            out_hbm.at[cid].at[pl.ds(r, zr)])

    return k(vals, idx3d)


_EDGE_BLOCK = 3200
_NODE_BLOCK = 2000


def _split_first(layers, widths):
    (w0, b0), l1, l2 = layers
    parts = []
    o = 0
    for w in widths:
        parts.append(w0[o:o + w])
        o += w
    return [(parts, b0), l1, l2]


def _pack_bf16(a):
    n, d2 = a.shape
    b = a.astype(jnp.bfloat16).reshape(n, d2 // 2, 2)
    return lax.bitcast_convert_type(b, jnp.float32)


def kernel(x, edge_attr, receivers, senders, params):
    n_nodes = x.shape[0]
    n_edges = receivers.shape[0]
    lat = params["enc_node"][-1][0].shape[1]

    nw = _NC * _NS
    g_pad = -(-2 * n_edges // (nw * _GW)) * (nw * _GW)
    gather_idx = jnp.concatenate(
        [receivers, senders + n_nodes,
         jnp.zeros((g_pad - 2 * n_edges,), jnp.int32)]).reshape(1, g_pad)
    e_pad = -(-n_edges // (nw * _SW)) * (nw * _SW)
    acc_rows = -(-(n_nodes + 1) // (_NS * 128)) * (_NS * 128)
    scat_idx = jnp.concatenate(
        [receivers,
         jnp.full((e_pad - n_edges,), n_nodes, jnp.int32)]).reshape(
             nw, e_pad // (nw * _SW), _SW)

    enc_node = _split_first(params["enc_node"], [x.shape[1]])
    enc_edge = _split_first(params["enc_edge"], [edge_attr.shape[1]])
    dec = _split_first(params["dec"], [lat])

    x_lat = _mlp([_rows2d(x)], enc_node, ln=True, block=_NODE_BLOCK)
    e_lat = _mlp([_rows2d(edge_attr)], enc_edge, ln=True, block=_EDGE_BLOCK)

    pre_x = x_lat
    upd_e_prev = None

    for p in params["proc"]:
        (w0e, b0e), l1e, l2e = _split_first(p["edge"], [lat, lat, lat])
        (w0n, b0n), l1n, l2n = _split_first(p["node"], [lat, lat])

        proj = _project_pair(pre_x, w0e[1], w0e[2], _NODE_BLOCK)
        g = _sc_gather(proj, gather_idx)

        eparts = [_rows2d_view(e_lat, 0, n_edges)]
        ew0 = [w0e[0]]
        if upd_e_prev is not None:
            eparts.append(_rows2d_view(upd_e_prev, 0, n_edges))
            ew0.append(w0e[0])
        eparts.append(_rows2d_view(g, 0, n_edges))
        ew0.append(None)
        eparts.append(_rows2d_view(g, n_edges, n_edges))
        ew0.append(None)
        upd_e = _mlp(eparts, [(ew0, b0e), l1e, l2e], ln=True,
                     block=_EDGE_BLOCK, out_rows=e_pad)

        partials = _sc_segment_sum(upd_e, scat_idx, acc_rows)

        nparts = [_rows2d(pre_x), _rows3d(partials, 0), _rows3d(partials, 1)]
        nw0 = [w0n[0], w0n[1], w0n[1]]
        pre_x = _mlp(nparts, [(nw0, b0n), l1n, l2n], ln=True,
                     resid=_rows2d(x_lat), block=_NODE_BLOCK)
        upd_e_prev = upd_e

    return _mlp([_rows2d(pre_x)], dec, ln=False, block=_NODE_BLOCK)


# speedup vs baseline: 2.6940x; 1.4351x over previous
import functools

import jax
import jax.numpy as jnp
from jax import lax
from jax.experimental import pallas as pl
from jax.experimental.pallas import tpu as pltpu
from jax.experimental.pallas import tpu_sc as plsc

_NC = 2
_NS = 16
_LN_EPS = 1e-5


def _mlp_body(kinds, ln, has_resid, first_only, *refs):
    nparts = len(kinds)
    nw = sum(1 for f in kinds if f == 0)
    parts = refs[:nparts]
    wrefs = iter(refs[nparts:nparts + nw])
    w0s = [None if f == 2 else next(wrefs) for f in kinds]
    b0, w1, b1, w2, b2 = refs[nparts + nw:nparts + nw + 5]
    resid_ref = refs[nparts + nw + 5] if has_resid else None
    out_ref = refs[-1]

    acc = None
    for p, w in zip(parts, w0s):
        xv = p[...]
        if xv.ndim == 3:
            xv = xv[0]
        if xv.dtype != jnp.float32:
            xv = xv.astype(jnp.float32)
        if w is None:
            t = xv
        else:
            t = jnp.dot(xv, w[...], preferred_element_type=jnp.float32)
        acc = t if acc is None else acc + t
    y = acc + b0[...]
    if not first_only:
        h = jnp.maximum(y, 0.0)
        h = jnp.maximum(
            jnp.dot(h, w1[...], preferred_element_type=jnp.float32)
            + b1[...], 0.0)
        y = jnp.dot(h, w2[...], preferred_element_type=jnp.float32) + b2[...]
    if ln:
        m = jnp.mean(y, axis=-1, keepdims=True)
        yc = y - m
        v = jnp.mean(yc * yc, axis=-1, keepdims=True)
        y = yc * lax.rsqrt(v + _LN_EPS)
    if has_resid:
        r = resid_ref[...]
        if r.ndim == 3:
            r = r[0]
        y = y + r
    out_ref[...] = y


def _full_spec(arr):
    nd = arr.ndim
    return pl.BlockSpec(arr.shape, lambda i, _n=nd: (0,) * _n)


def _mlp(parts, layers, *, ln, resid=None, block, out_rows=None,
         first_only=False):
    arrays = [a for a, _ in parts]
    specs = [s(block) for _, s in parts]
    (w0_list, b0), (w1, b1), (w2, b2) = layers
    n_rows = parts[0][1].n_rows
    out_dim = b0.size if first_only else w2.shape[1]
    b0 = b0.reshape(1, -1)
    b1 = b1.reshape(1, -1)
    b2 = b2.reshape(1, -1)
    kinds = [2 if w is None else 0 for w in w0_list]
    flat_w0 = [w for w in w0_list if w is not None]
    inputs = list(arrays) + flat_w0 + [b0, w1, b1, w2, b2]
    in_specs = list(specs) + [_full_spec(w) for w in flat_w0] + [
        _full_spec(b0), _full_spec(w1), _full_spec(b1),
        _full_spec(w2), _full_spec(b2)]
    if resid is not None:
        arr, sfn = resid
        inputs.append(arr)
        in_specs.append(sfn(block))
    return pl.pallas_call(
        functools.partial(_mlp_body, kinds, ln, resid is not None,
                          first_only),
        grid=(n_rows // block,),
        in_specs=in_specs,
        out_specs=pl.BlockSpec((block, out_dim), lambda i: (i, 0)),
        out_shape=jax.ShapeDtypeStruct((out_rows or n_rows, out_dim),
                                       jnp.float32),
    )(*inputs)


def _project_pair(x, wa, wb, block, n_pad):
    n, d = x.shape
    w = jnp.stack([wa, wb])
    nb = n // block

    def body(x_ref, w_ref, o_ref):
        o_ref[...] = jnp.dot(x_ref[...], w_ref[0],
                             preferred_element_type=jnp.float32)[None]

    return pl.pallas_call(
        body,
        grid=(2 * nb,),
        in_specs=[pl.BlockSpec((block, d), lambda i: (i % nb, 0)),
                  pl.BlockSpec((1,) + wa.shape, lambda i: (i // nb, 0, 0))],
        out_specs=pl.BlockSpec((1, block, wa.shape[1]),
                               lambda i: (i // nb, i % nb, 0)),
        out_shape=jax.ShapeDtypeStruct((2, n_pad, wa.shape[1]), jnp.float32),
    )(x, w)


def _rows2d(arr):
    d = arr.shape[1]

    def f(block):
        return pl.BlockSpec((block, d), lambda i: (i, 0))
    f.n_rows = arr.shape[0]
    return arr, f


def _rows2d_view(arr, row_offset, n_rows):
    d = arr.shape[1]

    def f(block):
        o = row_offset // block
        return pl.BlockSpec((block, d), lambda i, o=o: (o + i, 0))
    f.n_rows = n_rows
    return arr, f


def _rows3d(arr, major_idx):
    d = arr.shape[2]

    def f(block):
        return pl.BlockSpec((1, block, d), lambda i, m=major_idx: (m, i, 0))
    f.n_rows = arr.shape[1]
    return arr, f


_GW = 128


def _sc_gather(table, idx2d):
    n = idx2d.shape[1]
    d = table.shape[1]
    mesh = plsc.VectorSubcoreMesh(core_axis_name="core",
                                  subcore_axis_name="subcore")

    @functools.partial(
        pl.kernel,
        out_type=jax.ShapeDtypeStruct((n, d), table.dtype),
        mesh=mesh)
    def k(x_hbm, i_hbm, o_hbm):
        def body(i_vmem, o_vmem):
            pltpu.sync_copy(x_hbm.at[i_vmem.at[0]], o_vmem)

        pltpu.emit_pipeline(
            body,
            grid=(n // _GW,),
            in_specs=[pl.BlockSpec((1, _GW), lambda i: (0, i))],
            out_specs=[pl.BlockSpec((_GW, d), lambda i: (i, 0))],
            core_axis_name=("core", "subcore"),
            dimension_semantics=(pltpu.PARALLEL,),
        )(i_hbm, o_hbm)

    return k(table, idx2d)


def _sc_gather_spmem(tables, idx3d):
    _, n_rows, d = tables.shape
    half = idx3d.shape[2]
    rps = n_rows // _NS
    mesh = plsc.VectorSubcoreMesh(core_axis_name="core",
                                  subcore_axis_name="subcore")

    @functools.partial(
        pl.kernel,
        out_type=jax.ShapeDtypeStruct((_NC, half, d), jnp.float32),
        mesh=mesh,
        scratch_types=[pltpu.VMEM_SHARED((n_rows, d), jnp.float32)])
    def k(t_hbm, i_hbm, o_hbm, spm):
        cid = lax.axis_index("core")
        sid = lax.axis_index("subcore")
        pltpu.sync_copy(t_hbm.at[cid].at[pl.ds(sid * rps, rps)],
                        spm.at[pl.ds(sid * rps, rps)])
        plsc.subcore_barrier()

        def body(i_vmem, o_vmem):
            pltpu.sync_copy(spm.at[i_vmem.at[0]], o_vmem)

        pltpu.emit_pipeline(
            body,
            grid=(half // _GW,),
            in_specs=[pl.BlockSpec((1, _GW), lambda i: (0, i))],
            out_specs=[pl.BlockSpec((_GW, d), lambda i: (i, 0))],
            core_axis_name="subcore",
            dimension_semantics=(pltpu.PARALLEL,),
        )(i_hbm.at[cid], o_hbm.at[cid])

    return k(tables, idx3d)


_SW = 128


def _sc_segment_sum(vals, idx3d, acc_rows):
    d = vals.shape[1]
    k_chunks = idx3d.shape[1]
    per_w = k_chunks * _SW
    rows_per_sub = acc_rows // _NS
    zr = 128
    nz = rows_per_sub // zr
    mesh = plsc.VectorSubcoreMesh(core_axis_name="core",
                                  subcore_axis_name="subcore")

    @functools.partial(
        pl.kernel,
        out_type=jax.ShapeDtypeStruct((_NC, acc_rows, d), jnp.float32),
        mesh=mesh,
        scratch_types=[
            pltpu.VMEM((k_chunks, _SW), jnp.int32),
            pltpu.VMEM((_SW, d), jnp.float32),
            pltpu.VMEM((zr, d), jnp.float32),
            pltpu.VMEM_SHARED((acc_rows, d), jnp.float32),
        ])
    def k(vals_hbm, idx_hbm, out_hbm, idx_v, rows_v, zbuf, acc):
        cid = lax.axis_index("core")
        sid = lax.axis_index("subcore")
        wid = sid * _NC + cid

        zbuf[...] = jnp.zeros_like(zbuf)

        @pl.loop(0, nz)
        def _zero(z):
            pltpu.sync_copy(zbuf,
                            acc.at[pl.ds(sid * rows_per_sub + z * zr, zr)])

        plsc.subcore_barrier()

        pltpu.sync_copy(idx_hbm.at[wid], idx_v)

        @pl.loop(0, k_chunks)
        def _scat(j):
            base = wid * per_w + j * _SW
            pltpu.sync_copy(vals_hbm.at[pl.ds(base, _SW)], rows_v)
            pltpu.sync_copy(rows_v, acc.at[idx_v.at[j]], add=True)

        plsc.subcore_barrier()

        @pl.loop(0, nz)
        def _out(z):
            r = sid * rows_per_sub + z * zr
            pltpu.sync_copy(acc.at[pl.ds(r, zr)],
                            out_hbm.at[cid].at[pl.ds(r, zr)])

    return k(vals, idx3d)


_EDGE_BLOCK = 3200
_NODE_BLOCK = 2000


def _split_first(layers, widths):
    (w0, b0), l1, l2 = layers
    parts = []
    o = 0
    for w in widths:
        parts.append(w0[o:o + w])
        o += w
    return [(parts, b0), l1, l2]


def _pack_bf16(a):
    n, d2 = a.shape
    b = a.astype(jnp.bfloat16).reshape(n, d2 // 2, 2)
    return lax.bitcast_convert_type(b, jnp.float32)


def kernel(x, edge_attr, receivers, senders, params):
    n_nodes = x.shape[0]
    n_edges = receivers.shape[0]
    lat = params["enc_node"][-1][0].shape[1]

    nw = _NC * _NS
    n_pad = -(-n_nodes // (_NS * 8)) * (_NS * 8)
    g_half = -(-n_edges // (_NS * _GW)) * (_NS * _GW)
    zpad = jnp.zeros((g_half - n_edges,), jnp.int32)
    gather_idx = jnp.stack(
        [jnp.concatenate([receivers, zpad]),
         jnp.concatenate([senders, zpad])]).reshape(_NC, 1, g_half)
    e_pad = -(-n_edges // (nw * _SW)) * (nw * _SW)
    acc_rows = -(-(n_nodes + 1) // (_NS * 128)) * (_NS * 128)
    scat_idx = jnp.concatenate(
        [receivers,
         jnp.full((e_pad - n_edges,), n_nodes, jnp.int32)]).reshape(
             nw, e_pad // (nw * _SW), _SW)

    enc_node = _split_first(params["enc_node"], [x.shape[1]])
    enc_edge = _split_first(params["enc_edge"], [edge_attr.shape[1]])
    dec = _split_first(params["dec"], [lat])

    x_lat = _mlp([_rows2d(x)], enc_node, ln=True, block=_NODE_BLOCK)
    e_lat = _mlp([_rows2d(edge_attr)], enc_edge, ln=True, block=_EDGE_BLOCK)

    pre_x = x_lat
    upd_e_prev = None

    for p in params["proc"]:
        (w0e, b0e), l1e, l2e = _split_first(p["edge"], [lat, lat, lat])
        (w0n, b0n), l1n, l2n = _split_first(p["node"], [lat, lat])

        proj = _project_pair(pre_x, w0e[1], w0e[2], _NODE_BLOCK, n_pad)
        g = _sc_gather_spmem(proj, gather_idx)

        eparts = [_rows2d_view(e_lat, 0, n_edges)]
        ew0 = [w0e[0]]
        if upd_e_prev is not None:
            eparts.append(_rows2d_view(upd_e_prev, 0, n_edges))
            ew0.append(w0e[0])
        eparts.append(_rows3d(g, 0))
        ew0.append(None)
        eparts.append(_rows3d(g, 1))
        ew0.append(None)
        upd_e = _mlp(eparts, [(ew0, b0e), l1e, l2e], ln=True,
                     block=_EDGE_BLOCK, out_rows=e_pad)

        partials = _sc_segment_sum(upd_e, scat_idx, acc_rows)

        nparts = [_rows2d(pre_x), _rows3d(partials, 0), _rows3d(partials, 1)]
        nw0 = [w0n[0], w0n[1], w0n[1]]
        pre_x = _mlp(nparts, [(nw0, b0n), l1n, l2n], ln=True,
                     resid=_rows2d(x_lat), block=_NODE_BLOCK)
        upd_e_prev = upd_e

    return _mlp([_rows2d(pre_x)], dec, ln=False, block=_NODE_BLOCK)


# speedup vs baseline: 4.5081x; 1.6734x over previous
import functools

import jax
import jax.numpy as jnp
from jax import lax
from jax.experimental import pallas as pl
from jax.experimental.pallas import tpu as pltpu
from jax.experimental.pallas import tpu_sc as plsc

_NC = 2
_NS = 16
_LN_EPS = 1e-5


def _mlp_body(kinds, ln, has_resid, first_only, *refs):
    nparts = len(kinds)
    nw = sum(1 for f in kinds if f == 0)
    parts = refs[:nparts]
    wrefs = iter(refs[nparts:nparts + nw])
    w0s = [None if f == 2 else next(wrefs) for f in kinds]
    b0, w1, b1, w2, b2 = refs[nparts + nw:nparts + nw + 5]
    resid_ref = refs[nparts + nw + 5] if has_resid else None
    out_ref = refs[-1]

    acc = None
    for p, w in zip(parts, w0s):
        xv = p[...]
        if xv.ndim == 3:
            xv = xv[0]
        if xv.dtype != jnp.float32:
            xv = xv.astype(jnp.float32)
        if w is None:
            t = xv
        else:
            t = jnp.dot(xv, w[...], preferred_element_type=jnp.float32)
        acc = t if acc is None else acc + t
    y = acc + b0[...]
    if not first_only:
        h = jnp.maximum(y, 0.0)
        h = jnp.maximum(
            jnp.dot(h, w1[...], preferred_element_type=jnp.float32)
            + b1[...], 0.0)
        y = jnp.dot(h, w2[...], preferred_element_type=jnp.float32) + b2[...]
    if ln:
        m = jnp.mean(y, axis=-1, keepdims=True)
        yc = y - m
        v = jnp.mean(yc * yc, axis=-1, keepdims=True)
        y = yc * lax.rsqrt(v + _LN_EPS)
    if has_resid:
        r = resid_ref[...]
        if r.ndim == 3:
            r = r[0]
        y = y + r
    out_ref[...] = y


def _full_spec(arr):
    nd = arr.ndim
    return pl.BlockSpec(arr.shape, lambda i, _n=nd: (0,) * _n)


def _mlp(parts, layers, *, ln, resid=None, block, out_rows=None,
         first_only=False):
    arrays = [a for a, _ in parts]
    specs = [s(block) for _, s in parts]
    (w0_list, b0), (w1, b1), (w2, b2) = layers
    n_rows = parts[0][1].n_rows
    out_dim = b0.size if first_only else w2.shape[1]
    b0 = b0.reshape(1, -1)
    b1 = b1.reshape(1, -1)
    b2 = b2.reshape(1, -1)
    kinds = [2 if w is None else 0 for w in w0_list]
    flat_w0 = [w for w in w0_list if w is not None]
    inputs = list(arrays) + flat_w0 + [b0, w1, b1, w2, b2]
    in_specs = list(specs) + [_full_spec(w) for w in flat_w0] + [
        _full_spec(b0), _full_spec(w1), _full_spec(b1),
        _full_spec(w2), _full_spec(b2)]
    if resid is not None:
        arr, sfn = resid
        inputs.append(arr)
        in_specs.append(sfn(block))
    return pl.pallas_call(
        functools.partial(_mlp_body, kinds, ln, resid is not None,
                          first_only),
        grid=(n_rows // block,),
        in_specs=in_specs,
        out_specs=pl.BlockSpec((block, out_dim), lambda i: (i, 0)),
        out_shape=jax.ShapeDtypeStruct((out_rows or n_rows, out_dim),
                                       jnp.float32),
    )(*inputs)


def _project_pair(x, wa, wb, block, n_pad):
    n, d = x.shape
    w = jnp.stack([wa, wb])
    nb = n // block

    def body(x_ref, w_ref, o_ref):
        o_ref[...] = jnp.dot(x_ref[...], w_ref[0],
                             preferred_element_type=jnp.float32)[None]

    return pl.pallas_call(
        body,
        grid=(2 * nb,),
        in_specs=[pl.BlockSpec((block, d), lambda i: (i % nb, 0)),
                  pl.BlockSpec((1,) + wa.shape, lambda i: (i // nb, 0, 0))],
        out_specs=pl.BlockSpec((1, block, wa.shape[1]),
                               lambda i: (i // nb, i % nb, 0)),
        out_shape=jax.ShapeDtypeStruct((2, n_pad, wa.shape[1]), jnp.float32),
    )(x, w)


def _rows2d(arr):
    d = arr.shape[1]

    def f(block):
        return pl.BlockSpec((block, d), lambda i: (i, 0))
    f.n_rows = arr.shape[0]
    return arr, f


def _rows2d_view(arr, row_offset, n_rows):
    d = arr.shape[1]

    def f(block):
        o = row_offset // block
        return pl.BlockSpec((block, d), lambda i, o=o: (o + i, 0))
    f.n_rows = n_rows
    return arr, f


def _rows3d(arr, major_idx):
    d = arr.shape[2]

    def f(block):
        return pl.BlockSpec((1, block, d), lambda i, m=major_idx: (m, i, 0))
    f.n_rows = arr.shape[1]
    return arr, f


_GW = 128


def _sc_gather(table, idx2d):
    n = idx2d.shape[1]
    d = table.shape[1]
    mesh = plsc.VectorSubcoreMesh(core_axis_name="core",
                                  subcore_axis_name="subcore")

    @functools.partial(
        pl.kernel,
        out_type=jax.ShapeDtypeStruct((n, d), table.dtype),
        mesh=mesh)
    def k(x_hbm, i_hbm, o_hbm):
        def body(i_vmem, o_vmem):
            pltpu.sync_copy(x_hbm.at[i_vmem.at[0]], o_vmem)

        pltpu.emit_pipeline(
            body,
            grid=(n // _GW,),
            in_specs=[pl.BlockSpec((1, _GW), lambda i: (0, i))],
            out_specs=[pl.BlockSpec((_GW, d), lambda i: (i, 0))],
            core_axis_name=("core", "subcore"),
            dimension_semantics=(pltpu.PARALLEL,),
        )(i_hbm, o_hbm)

    return k(table, idx2d)


def _sc_gather_spmem(tables, idx3d):
    _, n_rows, d = tables.shape
    half = idx3d.shape[2]
    rps = n_rows // _NS
    mesh = plsc.VectorSubcoreMesh(core_axis_name="core",
                                  subcore_axis_name="subcore")

    @functools.partial(
        pl.kernel,
        out_type=jax.ShapeDtypeStruct((_NC, half, d), jnp.float32),
        mesh=mesh,
        scratch_types=[pltpu.VMEM_SHARED((n_rows, d), jnp.float32)])
    def k(t_hbm, i_hbm, o_hbm, spm):
        cid = lax.axis_index("core")
        sid = lax.axis_index("subcore")
        pltpu.sync_copy(t_hbm.at[cid].at[pl.ds(sid * rps, rps)],
                        spm.at[pl.ds(sid * rps, rps)])
        plsc.subcore_barrier()

        def body(i_vmem, o_vmem):
            pltpu.sync_copy(spm.at[i_vmem.at[0]], o_vmem)

        pltpu.emit_pipeline(
            body,
            grid=(half // _GW,),
            in_specs=[pl.BlockSpec((1, _GW), lambda i: (0, i))],
            out_specs=[pl.BlockSpec((_GW, d), lambda i: (i, 0))],
            core_axis_name="subcore",
            dimension_semantics=(pltpu.PARALLEL,),
        )(i_hbm.at[cid], o_hbm.at[cid])

    return k(tables, idx3d)


_SW = 128


def _sc_segment_sum(vals3, idx3, acc_rows):
    _, eh, d = vals3.shape
    rows_per_sub = acc_rows // _NS
    zr = 64
    nz = rows_per_sub // zr
    ocr = 128
    no = rows_per_sub // ocr
    mesh = plsc.VectorSubcoreMesh(core_axis_name="core",
                                  subcore_axis_name="subcore")

    @functools.partial(
        pl.kernel,
        out_type=jax.ShapeDtypeStruct((_NC, acc_rows, d), jnp.float32),
        mesh=mesh,
        scratch_types=[
            pltpu.VMEM((zr, d), jnp.float32),
            pltpu.VMEM_SHARED((acc_rows, d), jnp.float32),
        ])
    def k(vals_hbm, idx_hbm, out_hbm, zbuf, acc):
        cid = lax.axis_index("core")
        sid = lax.axis_index("subcore")

        zbuf[...] = jnp.zeros_like(zbuf)

        @pl.loop(0, nz)
        def _zero(z):
            pltpu.sync_copy(zbuf,
                            acc.at[pl.ds(sid * rows_per_sub + z * zr, zr)])

        plsc.subcore_barrier()

        def body(v_vmem, i_vmem):
            pltpu.sync_copy(v_vmem, acc.at[i_vmem.at[0]], add=True)

        pltpu.emit_pipeline(
            body,
            grid=(eh // _SW,),
            in_specs=[pl.BlockSpec((_SW, d), lambda i: (i, 0)),
                      pl.BlockSpec((1, _SW), lambda i: (0, i))],
            out_specs=[],
            core_axis_name="subcore",
            dimension_semantics=(pltpu.PARALLEL,),
        )(vals_hbm.at[cid], idx_hbm.at[cid])

        plsc.subcore_barrier()

        @pl.loop(0, no)
        def _out(z):
            r = sid * rows_per_sub + z * ocr
            pltpu.sync_copy(acc.at[pl.ds(r, ocr)],
                            out_hbm.at[cid].at[pl.ds(r, ocr)])

    return k(vals3, idx3)


_EDGE_BLOCK = 3200
_NODE_BLOCK = 2000


def _split_first(layers, widths):
    (w0, b0), l1, l2 = layers
    parts = []
    o = 0
    for w in widths:
        parts.append(w0[o:o + w])
        o += w
    return [(parts, b0), l1, l2]


def _pack_bf16(a):
    n, d2 = a.shape
    b = a.astype(jnp.bfloat16).reshape(n, d2 // 2, 2)
    return lax.bitcast_convert_type(b, jnp.float32)


def kernel(x, edge_attr, receivers, senders, params):
    n_nodes = x.shape[0]
    n_edges = receivers.shape[0]
    lat = params["enc_node"][-1][0].shape[1]

    nw = _NC * _NS
    n_pad = -(-n_nodes // (_NS * 8)) * (_NS * 8)
    g_half = -(-n_edges // (_NS * _GW)) * (_NS * _GW)
    zpad = jnp.zeros((g_half - n_edges,), jnp.int32)
    gather_idx = jnp.stack(
        [jnp.concatenate([receivers, zpad]),
         jnp.concatenate([senders, zpad])]).reshape(_NC, 1, g_half)
    e_pad = -(-n_edges // (nw * _SW)) * (nw * _SW)
    acc_rows = -(-(n_nodes + 1) // (_NS * 128)) * (_NS * 128)
    scat_idx = jnp.concatenate(
        [receivers,
         jnp.full((e_pad - n_edges,), n_nodes, jnp.int32)]).reshape(
             _NC, 1, e_pad // _NC)

    enc_node = _split_first(params["enc_node"], [x.shape[1]])
    enc_edge = _split_first(params["enc_edge"], [edge_attr.shape[1]])
    dec = _split_first(params["dec"], [lat])

    x_lat = _mlp([_rows2d(x)], enc_node, ln=True, block=_NODE_BLOCK)
    e_lat = _mlp([_rows2d(edge_attr)], enc_edge, ln=True, block=_EDGE_BLOCK)

    pre_x = x_lat
    upd_e_prev = None

    for p in params["proc"]:
        (w0e, b0e), l1e, l2e = _split_first(p["edge"], [lat, lat, lat])
        (w0n, b0n), l1n, l2n = _split_first(p["node"], [lat, lat])

        proj = _project_pair(pre_x, w0e[1], w0e[2], _NODE_BLOCK, n_pad)
        g = _sc_gather_spmem(proj, gather_idx)

        eparts = [_rows2d_view(e_lat, 0, n_edges)]
        ew0 = [w0e[0]]
        if upd_e_prev is not None:
            eparts.append(_rows2d_view(upd_e_prev, 0, n_edges))
            ew0.append(w0e[0])
        eparts.append(_rows3d(g, 0))
        ew0.append(None)
        eparts.append(_rows3d(g, 1))
        ew0.append(None)
        upd_e = _mlp(eparts, [(ew0, b0e), l1e, l2e], ln=True,
                     block=_EDGE_BLOCK, out_rows=e_pad)

        partials = _sc_segment_sum(
            upd_e.reshape(_NC, e_pad // _NC, lat), scat_idx, acc_rows)

        nparts = [_rows2d(pre_x), _rows3d(partials, 0), _rows3d(partials, 1)]
        nw0 = [w0n[0], w0n[1], w0n[1]]
        pre_x = _mlp(nparts, [(nw0, b0n), l1n, l2n], ln=True,
                     resid=_rows2d(x_lat), block=_NODE_BLOCK)
        upd_e_prev = upd_e

    return _mlp([_rows2d(pre_x)], dec, ln=False, block=_NODE_BLOCK)
